# baseline (speedup 1.0000x reference)
import functools

import jax
import jax.numpy as jnp
from jax import lax
from jax.experimental import pallas as pl
from jax.experimental.pallas import tpu as pltpu
from jax.experimental.pallas import tpu_sc as plsc

N_STYLE = 1000000
D_STYLE = 64
BATCH = 16384
HIST = 20

NC = 2
NS = 16
NW = NC * NS

NIDX = BATCH * HIST
N_PER_W = NIDX // NW
CHUNK = 512
N_CHUNKS = N_PER_W // CHUNK


def _gather_body(idx_hbm, table_hbm, out_hbm, idx_v, rows_v, sem):
    wid = lax.axis_index("s") * NC + lax.axis_index("c")
    base = wid * N_PER_W
    pltpu.sync_copy(idx_hbm.at[pl.ds(base, N_PER_W)], idx_v)

    def chunk_step(j, carry):
        off = j * CHUNK
        idx_chunk = idx_v.at[pl.ds(off, CHUNK)]
        pltpu.async_copy(table_hbm.at[idx_chunk], rows_v, sem).wait()
        pltpu.sync_copy(rows_v, out_hbm.at[pl.ds(base + off, CHUNK)])
        return carry

    lax.fori_loop(0, N_CHUNKS, chunk_step, 0)


@jax.jit
def _embed(x_flat, lut):
    mesh = plsc.VectorSubcoreMesh(
        core_axis_name="c", subcore_axis_name="s", num_cores=NC,
        num_subcores=NS)
    f = pl.kernel(
        _gather_body,
        out_type=jax.ShapeDtypeStruct((NIDX, D_STYLE), jnp.float32),
        mesh=mesh,
        scratch_types=[
            pltpu.VMEM((N_PER_W,), jnp.int32),
            pltpu.VMEM((CHUNK, D_STYLE), jnp.float32),
            pltpu.SemaphoreType.DMA,
        ],
        compiler_params=pltpu.CompilerParams(use_tc_tiling_on_sc=False),
    )
    return f(x_flat, lut)


def kernel(x, lut):
    x_flat = x.reshape(NIDX).astype(jnp.int32)
    out = _embed(x_flat, lut)
    return out.reshape(BATCH, HIST, D_STYLE)
